# baseline (speedup 1.0000x reference)
import jax
import jax.numpy as jnp
from jax.experimental import pallas as pl

NUM_CLASSES = 1000
ROWS = 4096 * 26
BLOCK_ROWS = 512


def _onehot_body(x_ref, o_ref):
    idx = x_ref[0, 0, :]
    iota = jax.lax.broadcasted_iota(jnp.int32, (BLOCK_ROWS, NUM_CLASSES), 1)
    o_ref[...] = (iota == idx[:, None]).astype(jnp.int32)


def kernel(x):
    n_blocks = ROWS // BLOCK_ROWS
    x_flat = x.reshape(n_blocks, 1, BLOCK_ROWS)
    out = pl.pallas_call(
        _onehot_body,
        grid=(n_blocks,),
        in_specs=[pl.BlockSpec((1, 1, BLOCK_ROWS), lambda i: (i, 0, 0))],
        out_specs=pl.BlockSpec((BLOCK_ROWS, NUM_CLASSES), lambda i: (i, 0)),
        out_shape=jax.ShapeDtypeStruct((ROWS, NUM_CLASSES), jnp.int32),
    )(x_flat)
    return out.reshape(4096, 26, NUM_CLASSES)


# speedup vs baseline: 1.4580x; 1.4580x over previous
import jax
import jax.numpy as jnp
from jax.experimental import pallas as pl

NUM_CLASSES = 1000
B0 = 4096
B1 = 26
BLOCK = 32


def _onehot_body(x_ref, o_ref):
    idx = x_ref[...]
    iota = jax.lax.broadcasted_iota(jnp.int32, (BLOCK, B1, NUM_CLASSES), 2)
    o_ref[...] = (iota == idx[:, :, None]).astype(jnp.int32)


def kernel(x):
    out = pl.pallas_call(
        _onehot_body,
        grid=(B0 // BLOCK,),
        in_specs=[pl.BlockSpec((BLOCK, B1), lambda i: (i, 0))],
        out_specs=pl.BlockSpec((BLOCK, B1, NUM_CLASSES), lambda i: (i, 0, 0)),
        out_shape=jax.ShapeDtypeStruct((B0, B1, NUM_CLASSES), jnp.int32),
    )(x)
    return out
